# Initial kernel scaffold; baseline (speedup 1.0000x reference)
import functools

import jax
import jax.numpy as jnp
from jax import lax
from jax.experimental import pallas as pl
from jax.experimental.pallas import tpu as pltpu
from jax.experimental.pallas import tpu_sc as plsc

N = 50000
NPAD = 50176
M = 250
MPAD = 256
K = 16
BLK = 3584
NBLK = NPAD // BLK
PADV = jnp.float32(1e30)
MASKV = jnp.float32(2e30)
IBIG = jnp.int32(2**30)


def _topk_kernel(q_ref, qn_ref, pt_ref, pn_ref, out_ref, vals_ref, inds_ref):
    j = pl.program_id(0)

    @pl.when(j == 0)
    def _init():
        vals_ref[...] = jnp.full((MPAD, K), MASKV, jnp.float32)
        inds_ref[...] = jnp.zeros((MPAD, K), jnp.int32)

    q = q_ref[...]
    qn = qn_ref[...]
    pt = pt_ref[...]
    pn = pn_ref[...]
    mm = jnp.dot(q, pt)
    d = (qn - 2.0 * mm) + pn
    cols = lax.broadcasted_iota(jnp.int32, (MPAD, BLK), 1) + j * BLK

    svals = vals_ref[...]
    sinds = inds_ref[...]

    new_vals = []
    new_inds = []
    for _ in range(K):
        m_d = jnp.min(d, axis=1, keepdims=True)
        m_s = jnp.min(svals, axis=1, keepdims=True)
        m = jnp.minimum(m_d, m_s)
        eq_d = d == m
        eq_s = svals == m
        c_d = jnp.min(jnp.where(eq_d, cols, IBIG), axis=1, keepdims=True)
        c_s = jnp.min(jnp.where(eq_s, sinds, IBIG), axis=1, keepdims=True)
        sel = jnp.minimum(c_d, c_s)
        new_vals.append(m)
        new_inds.append(sel)
        d = jnp.where(eq_d & (cols == sel), MASKV, d)
        svals = jnp.where(eq_s & (sinds == sel), MASKV, svals)

    vals_ref[...] = jnp.concatenate(new_vals, axis=1)
    inds_ref[...] = jnp.concatenate(new_inds, axis=1)

    @pl.when(j == NBLK - 1)
    def _emit():
        out_ref[...] = jnp.concatenate(new_inds, axis=1)


_topk_call = pl.pallas_call(
    _topk_kernel,
    grid=(NBLK,),
    in_specs=[
        pl.BlockSpec((MPAD, 3), lambda j: (0, 0)),
        pl.BlockSpec((MPAD, 1), lambda j: (0, 0)),
        pl.BlockSpec((3, BLK), lambda j: (0, j)),
        pl.BlockSpec((1, BLK), lambda j: (0, j)),
    ],
    out_specs=pl.BlockSpec((MPAD, K), lambda j: (0, 0)),
    out_shape=jax.ShapeDtypeStruct((MPAD, K), jnp.int32),
    scratch_shapes=[
        pltpu.VMEM((MPAD, K), jnp.float32),
        pltpu.VMEM((MPAD, K), jnp.int32),
    ],
    compiler_params=pltpu.CompilerParams(
        dimension_semantics=("arbitrary",),
    ),
)


_NW = 32
_BG = MPAD * K
_BPW = _BG // _NW
_D = 128

_sc_mesh = plsc.VectorSubcoreMesh(core_axis_name="c", subcore_axis_name="s")


@functools.partial(
    pl.kernel,
    out_type=jax.ShapeDtypeStruct((_BG, _D), jnp.float32),
    mesh=_sc_mesh,
    scratch_types=[
        pltpu.VMEM((_BPW,), jnp.int32),
        pltpu.VMEM((_BPW, _D), jnp.float32),
        pltpu.SemaphoreType.DMA,
    ],
)
def _gather_call(table_hbm, idx_hbm, out_hbm, idx_v, rows_v, sem):
    wid = lax.axis_index("s") * 2 + lax.axis_index("c")
    base = wid * _BPW
    pltpu.sync_copy(idx_hbm.at[pl.ds(base, _BPW)], idx_v)
    pltpu.async_copy(table_hbm.at[idx_v], rows_v, sem).wait()
    pltpu.sync_copy(rows_v, out_hbm.at[pl.ds(base, _BPW)])


def kernel(feats):
    p = feats[:, :3]
    n_p = p[:M]
    qn = jnp.sum(n_p[:, None, :] ** 2, axis=-1, keepdims=True)
    pn = jnp.sum(p ** 2, axis=-1)

    q_pad = jnp.zeros((MPAD, 3), jnp.float32).at[:M].set(n_p)
    qn_pad = jnp.zeros((MPAD, 1), jnp.float32).at[:M].set(qn.reshape(M, 1))
    pt_pad = jnp.pad(p.T, ((0, 0), (0, NPAD - N)))
    pn_pad = jnp.pad(pn, (0, NPAD - N), constant_values=PADV).reshape(1, NPAD)

    idx = _topk_call(q_pad, qn_pad, pt_pad, pn_pad)

    rows = _gather_call(feats, idx.reshape(_BG))
    g = rows.reshape(MPAD, K, _D)[:M]
    out = jnp.concatenate([g[:, :, :3] - n_p[:, None, :], g[:, :, 3:]], axis=-1)
    return out


# speedup vs baseline: 742.4060x; 742.4060x over previous
import functools

import jax
import jax.numpy as jnp
from jax import lax
from jax.experimental import pallas as pl
from jax.experimental.pallas import tpu as pltpu
from jax.experimental.pallas import tpu_sc as plsc

N = 50000
NPAD = 50176
M = 250
MPAD = 256
K = 16
C = 32
BLK = 7168
NBLK = NPAD // BLK
PADV = 1e30
MASKV = 2e30
IBIG = 2**30


def _short_kernel(q_ref, qn_ref, pt_ref, pn_ref, cidx_ref, cmm_ref,
                  vals_ref, inds_ref, mms_ref):
    j = pl.program_id(0)

    @pl.when(j == 0)
    def _init():
        vals_ref[...] = jnp.full((MPAD, C), MASKV, jnp.float32)
        inds_ref[...] = jnp.zeros((MPAD, C), jnp.int32)
        mms_ref[...] = jnp.zeros((MPAD, C), jnp.float32)

    q = q_ref[...]
    qn = qn_ref[...]
    pt = pt_ref[...]
    pn = pn_ref[...]
    mm = jnp.dot(q, pt)
    d = (qn - 2.0 * mm) + pn
    cols = lax.broadcasted_iota(jnp.int32, (MPAD, BLK), 1) + j * BLK

    svals = vals_ref[...]
    sinds = inds_ref[...]
    smms = mms_ref[...]

    new_vals, new_inds, new_mms = [], [], []
    for _ in range(C):
        m_d = jnp.min(d, axis=1, keepdims=True)
        m_s = jnp.min(svals, axis=1, keepdims=True)
        m = jnp.minimum(m_d, m_s)
        eq_d = d == m
        eq_s = svals == m
        c_d = jnp.min(jnp.where(eq_d, cols, IBIG), axis=1, keepdims=True)
        c_s = jnp.min(jnp.where(eq_s, sinds, IBIG), axis=1, keepdims=True)
        sel = jnp.minimum(c_d, c_s)
        hit_d = eq_d & (cols == sel)
        hit_s = eq_s & (sinds == sel)
        mm_d = jnp.min(jnp.where(hit_d, mm, MASKV), axis=1, keepdims=True)
        mm_s = jnp.min(jnp.where(hit_s, smms, MASKV), axis=1, keepdims=True)
        new_vals.append(m)
        new_inds.append(sel)
        new_mms.append(jnp.minimum(mm_d, mm_s))
        d = jnp.where(hit_d, MASKV, d)
        svals = jnp.where(hit_s, MASKV, svals)

    vals_ref[...] = jnp.concatenate(new_vals, axis=1)
    inds_ref[...] = jnp.concatenate(new_inds, axis=1)
    mms_ref[...] = jnp.concatenate(new_mms, axis=1)

    @pl.when(j == NBLK - 1)
    def _emit():
        cidx_ref[...] = jnp.concatenate(new_inds, axis=1)
        cmm_ref[...] = jnp.concatenate(new_mms, axis=1)


_short_call = pl.pallas_call(
    _short_kernel,
    grid=(NBLK,),
    in_specs=[
        pl.BlockSpec((MPAD, 3), lambda j: (0, 0)),
        pl.BlockSpec((MPAD, 1), lambda j: (0, 0)),
        pl.BlockSpec((3, BLK), lambda j: (0, j)),
        pl.BlockSpec((1, BLK), lambda j: (0, j)),
    ],
    out_specs=[
        pl.BlockSpec((MPAD, C), lambda j: (0, 0)),
        pl.BlockSpec((MPAD, C), lambda j: (0, 0)),
    ],
    out_shape=[
        jax.ShapeDtypeStruct((MPAD, C), jnp.int32),
        jax.ShapeDtypeStruct((MPAD, C), jnp.float32),
    ],
    scratch_shapes=[
        pltpu.VMEM((MPAD, C), jnp.float32),
        pltpu.VMEM((MPAD, C), jnp.int32),
        pltpu.VMEM((MPAD, C), jnp.float32),
    ],
    compiler_params=pltpu.CompilerParams(
        dimension_semantics=("arbitrary",),
    ),
)


_NW = 32
_BG = MPAD * C
_BPW = _BG // _NW
_D = 128


@functools.lru_cache(maxsize=1)
def _make_gather_call():
    mesh = plsc.VectorSubcoreMesh(core_axis_name="c", subcore_axis_name="s")

    @functools.partial(
        pl.kernel,
        out_type=jax.ShapeDtypeStruct((_BG, _D), jnp.float32),
        mesh=mesh,
        scratch_types=[
            pltpu.VMEM((_BPW,), jnp.int32),
            pltpu.VMEM((_BPW, _D), jnp.float32),
            pltpu.SemaphoreType.DMA,
        ],
    )
    def _gather_call(table_hbm, idx_hbm, out_hbm, idx_v, rows_v, sem):
        wid = lax.axis_index("s") * 2 + lax.axis_index("c")
        base = wid * _BPW
        pltpu.sync_copy(idx_hbm.at[pl.ds(base, _BPW)], idx_v)
        pltpu.async_copy(table_hbm.at[idx_v], rows_v, sem).wait()
        pltpu.sync_copy(rows_v, out_hbm.at[pl.ds(base, _BPW)])

    return _gather_call


def _asm_kernel(qn_ref, ci_ref, cm_ref, cp_ref, cand_ref, out_ref):
    qn = qn_ref[...]
    ci = ci_ref[...].reshape(1, C)
    cm = cm_ref[...].reshape(1, C)
    cp = cp_ref[...].reshape(1, C)
    cand = cand_ref[...].reshape(C, _D)
    d = (qn - 2.0 * cm) + cp
    for r in range(K):
        m = jnp.min(d, axis=1, keepdims=True)
        eq = d == m
        sel = jnp.min(jnp.where(eq, ci, IBIG), axis=1, keepdims=True)
        hit = eq & (ci == sel)
        oh = hit.astype(jnp.float32)
        row = jnp.dot(oh, cand, precision=jax.lax.Precision.HIGHEST)
        out_ref[:, 0, r, :] = row[:M]
        d = jnp.where(hit, MASKV, d)


_asm_call = pl.pallas_call(
    _asm_kernel,
    grid=(M,),
    in_specs=[
        pl.BlockSpec((MPAD, 1), lambda q: (0, 0)),
        pl.BlockSpec((1, 1, C), lambda q: (q, 0, 0)),
        pl.BlockSpec((1, 1, C), lambda q: (q, 0, 0)),
        pl.BlockSpec((1, 1, C), lambda q: (q, 0, 0)),
        pl.BlockSpec((1, C, _D), lambda q: (q, 0, 0)),
    ],
    out_specs=pl.BlockSpec((M, 1, K, _D), lambda q: (0, q, 0, 0)),
    out_shape=jax.ShapeDtypeStruct((M, M, K, _D), jnp.float32),
    compiler_params=pltpu.CompilerParams(
        dimension_semantics=("arbitrary",),
    ),
)


def kernel(feats):
    p = feats[:, :3]
    n_p = p[:M]
    qn = jnp.sum(n_p[:, None, :] ** 2, axis=-1, keepdims=True)
    pn = jnp.sum(p ** 2, axis=-1)

    q_pad = jnp.zeros((MPAD, 3), jnp.float32).at[:M].set(n_p)
    qn_pad = jnp.zeros((MPAD, 1), jnp.float32).at[:M].set(qn.reshape(M, 1))
    pt_pad = jnp.pad(p.T, ((0, 0), (0, NPAD - N)))
    pn_pad = jnp.pad(pn, (0, NPAD - N), constant_values=PADV).reshape(1, NPAD)

    cidx, cmm = _short_call(q_pad, qn_pad, pt_pad, pn_pad)

    rows = _make_gather_call()(feats, cidx.reshape(_BG))
    cand = rows.reshape(MPAD, C, _D)
    cand = jnp.concatenate(
        [cand[:, :, :3] - q_pad[:, None, :], cand[:, :, 3:]], axis=-1)

    cpn = jnp.take(pn, cidx)
    out = _asm_call(qn_pad, cidx[:, None, :], cmm[:, None, :],
                    cpn[:, None, :], cand)
    return out


# speedup vs baseline: 971.9995x; 1.3093x over previous
import functools

import jax
import jax.numpy as jnp
from jax import lax
from jax.experimental import pallas as pl
from jax.experimental.pallas import tpu as pltpu
from jax.experimental.pallas import tpu_sc as plsc

N = 50000
NPAD = 50176
M = 250
MPAD = 256
K = 16
C = 24
QB = 5
BLK = 7168
NBLK = NPAD // BLK
PADV = 1e30
MASKV = 2e30
IBIG = 2**30


def _short_kernel(q_ref, qn_ref, pt_ref, pn_ref, cidx_ref, cmm_ref,
                  vals_ref, inds_ref, mms_ref):
    j = pl.program_id(0)

    @pl.when(j == 0)
    def _init():
        vals_ref[...] = jnp.full((MPAD, C), MASKV, jnp.float32)
        inds_ref[...] = jnp.zeros((MPAD, C), jnp.int32)
        mms_ref[...] = jnp.zeros((MPAD, C), jnp.float32)

    q = q_ref[...]
    qn = qn_ref[...]
    pt = pt_ref[...]
    pn = pn_ref[...]
    mm = jnp.dot(q, pt)
    d = (qn - 2.0 * mm) + pn
    cols = lax.broadcasted_iota(jnp.int32, (MPAD, BLK), 1) + j * BLK

    svals = vals_ref[...]
    sinds = inds_ref[...]
    smms = mms_ref[...]

    new_vals, new_inds, new_mms = [], [], []
    for _ in range(C):
        m_d = jnp.min(d, axis=1, keepdims=True)
        m_s = jnp.min(svals, axis=1, keepdims=True)
        m = jnp.minimum(m_d, m_s)
        t_d = jnp.where(d == m, cols, IBIG)
        t_s = jnp.where(svals == m, sinds, IBIG)
        c_d = jnp.min(t_d, axis=1, keepdims=True)
        c_s = jnp.min(t_s, axis=1, keepdims=True)
        sel = jnp.minimum(c_d, c_s)
        hit_d = t_d == sel
        hit_s = t_s == sel
        mm_d = jnp.min(jnp.where(hit_d, mm, MASKV), axis=1, keepdims=True)
        mm_s = jnp.min(jnp.where(hit_s, smms, MASKV), axis=1, keepdims=True)
        new_vals.append(m)
        new_inds.append(sel)
        new_mms.append(jnp.minimum(mm_d, mm_s))
        d = jnp.where(hit_d, MASKV, d)
        svals = jnp.where(hit_s, MASKV, svals)

    vals_ref[...] = jnp.concatenate(new_vals, axis=1)
    inds_ref[...] = jnp.concatenate(new_inds, axis=1)
    mms_ref[...] = jnp.concatenate(new_mms, axis=1)

    @pl.when(j == NBLK - 1)
    def _emit():
        cidx_ref[...] = jnp.concatenate(new_inds, axis=1)
        cmm_ref[...] = jnp.concatenate(new_mms, axis=1)


_short_call = pl.pallas_call(
    _short_kernel,
    grid=(NBLK,),
    in_specs=[
        pl.BlockSpec((MPAD, 3), lambda j: (0, 0)),
        pl.BlockSpec((MPAD, 1), lambda j: (0, 0)),
        pl.BlockSpec((3, BLK), lambda j: (0, j)),
        pl.BlockSpec((1, BLK), lambda j: (0, j)),
    ],
    out_specs=[
        pl.BlockSpec((MPAD, C), lambda j: (0, 0)),
        pl.BlockSpec((MPAD, C), lambda j: (0, 0)),
    ],
    out_shape=[
        jax.ShapeDtypeStruct((MPAD, C), jnp.int32),
        jax.ShapeDtypeStruct((MPAD, C), jnp.float32),
    ],
    scratch_shapes=[
        pltpu.VMEM((MPAD, C), jnp.float32),
        pltpu.VMEM((MPAD, C), jnp.int32),
        pltpu.VMEM((MPAD, C), jnp.float32),
    ],
    compiler_params=pltpu.CompilerParams(
        dimension_semantics=("arbitrary",),
    ),
)


_NW = 32
_BG = MPAD * C
_BPW = _BG // _NW
_D = 128


@functools.lru_cache(maxsize=1)
def _make_gather_call():
    mesh = plsc.VectorSubcoreMesh(core_axis_name="c", subcore_axis_name="s")

    @functools.partial(
        pl.kernel,
        out_type=jax.ShapeDtypeStruct((_BG, _D), jnp.float32),
        mesh=mesh,
        scratch_types=[
            pltpu.VMEM((_BPW,), jnp.int32),
            pltpu.VMEM((_BPW, _D), jnp.float32),
            pltpu.SemaphoreType.DMA,
        ],
    )
    def _gather_call(table_hbm, idx_hbm, out_hbm, idx_v, rows_v, sem):
        wid = lax.axis_index("s") * 2 + lax.axis_index("c")
        base = wid * _BPW
        pltpu.sync_copy(idx_hbm.at[pl.ds(base, _BPW)], idx_v)
        pltpu.async_copy(table_hbm.at[idx_v], rows_v, sem).wait()
        pltpu.sync_copy(rows_v, out_hbm.at[pl.ds(base, _BPW)])

    return _gather_call


def _asm_kernel(qn_ref, ci_ref, cm_ref, cp_ref, cand_ref, out_ref):
    qn = qn_ref[...]
    for b in range(QB):
        ci = ci_ref[b].reshape(1, C)
        cm = cm_ref[b].reshape(1, C)
        cp = cp_ref[b].reshape(1, C)
        cand = cand_ref[b]
        d = (qn - 2.0 * cm) + cp
        for r in range(K):
            m = jnp.min(d, axis=1, keepdims=True)
            t = jnp.where(d == m, ci, IBIG)
            sel = jnp.min(t, axis=1, keepdims=True)
            hit = t == sel
            oh = hit.astype(jnp.float32)
            row = jnp.dot(oh, cand, precision=jax.lax.Precision.HIGHEST)
            out_ref[:, b, r, :] = row[:M]
            d = jnp.where(hit, MASKV, d)


_asm_call = pl.pallas_call(
    _asm_kernel,
    grid=(M // QB,),
    in_specs=[
        pl.BlockSpec((MPAD, 1), lambda q: (0, 0)),
        pl.BlockSpec((QB, 1, C), lambda q: (q, 0, 0)),
        pl.BlockSpec((QB, 1, C), lambda q: (q, 0, 0)),
        pl.BlockSpec((QB, 1, C), lambda q: (q, 0, 0)),
        pl.BlockSpec((QB, C, _D), lambda q: (q, 0, 0)),
    ],
    out_specs=pl.BlockSpec((M, QB, K, _D), lambda q: (0, q, 0, 0)),
    out_shape=jax.ShapeDtypeStruct((M, M, K, _D), jnp.float32),
    compiler_params=pltpu.CompilerParams(
        dimension_semantics=("arbitrary",),
    ),
)


def kernel(feats):
    p = feats[:, :3]
    n_p = p[:M]
    qn = jnp.sum(n_p[:, None, :] ** 2, axis=-1, keepdims=True)
    pn = jnp.sum(p ** 2, axis=-1)

    q_pad = jnp.zeros((MPAD, 3), jnp.float32).at[:M].set(n_p)
    qn_pad = jnp.zeros((MPAD, 1), jnp.float32).at[:M].set(qn.reshape(M, 1))
    pt_pad = jnp.pad(p.T, ((0, 0), (0, NPAD - N)))
    pn_pad = jnp.pad(pn, (0, NPAD - N), constant_values=PADV).reshape(1, NPAD)

    cidx, cmm = _short_call(q_pad, qn_pad, pt_pad, pn_pad)

    rows = _make_gather_call()(feats, cidx.reshape(_BG))
    cand = rows.reshape(MPAD, C, _D)
    cand = jnp.concatenate(
        [cand[:, :, :3] - q_pad[:, None, :], cand[:, :, 3:]], axis=-1)

    cpn = jnp.take(pn, cidx)
    out = _asm_call(qn_pad, cidx[:, None, :], cmm[:, None, :],
                    cpn[:, None, :], cand)
    return out


# speedup vs baseline: 981.6795x; 1.0100x over previous
import functools

import jax
import jax.numpy as jnp
from jax import lax
from jax.experimental import pallas as pl
from jax.experimental.pallas import tpu as pltpu
from jax.experimental.pallas import tpu_sc as plsc

N = 50000
NPAD = 50176
M = 250
MPAD = 256
K = 16
C = 24
QB = 5
BLK = 7168
NBLK = NPAD // BLK
PADV = 1e30
MASKV = 2e30
IBIG = 2**30


def _short_kernel(q_ref, qn_ref, pt_ref, pn_ref, cidx_ref, cmm_ref,
                  vals_ref, inds_ref, mms_ref):
    j = pl.program_id(0)

    @pl.when(j == 0)
    def _init():
        vals_ref[...] = jnp.full((MPAD, C), MASKV, jnp.float32)
        inds_ref[...] = jnp.zeros((MPAD, C), jnp.int32)
        mms_ref[...] = jnp.zeros((MPAD, C), jnp.float32)

    q = q_ref[...]
    qn = qn_ref[...]
    pt = pt_ref[...]
    pn = pn_ref[...]
    mm = jnp.dot(q, pt)
    d = (qn - 2.0 * mm) + pn
    cols = lax.broadcasted_iota(jnp.int32, (MPAD, BLK), 1) + j * BLK

    svals = vals_ref[...]
    sinds = inds_ref[...]
    smms = mms_ref[...]

    new_vals, new_inds, new_mms = [], [], []
    for _ in range(C):
        m_d = jnp.min(d, axis=1, keepdims=True)
        m_s = jnp.min(svals, axis=1, keepdims=True)
        m = jnp.minimum(m_d, m_s)
        t_d = jnp.where(d == m, cols, IBIG)
        t_s = jnp.where(svals == m, sinds, IBIG)
        c_d = jnp.min(t_d, axis=1, keepdims=True)
        c_s = jnp.min(t_s, axis=1, keepdims=True)
        sel = jnp.minimum(c_d, c_s)
        hit_d = t_d == sel
        hit_s = t_s == sel
        mm_d = jnp.min(jnp.where(hit_d, mm, MASKV), axis=1, keepdims=True)
        mm_s = jnp.min(jnp.where(hit_s, smms, MASKV), axis=1, keepdims=True)
        new_vals.append(m)
        new_inds.append(sel)
        new_mms.append(jnp.minimum(mm_d, mm_s))
        d = jnp.where(hit_d, MASKV, d)
        svals = jnp.where(hit_s, MASKV, svals)

    vals_ref[...] = jnp.concatenate(new_vals, axis=1)
    inds_ref[...] = jnp.concatenate(new_inds, axis=1)
    mms_ref[...] = jnp.concatenate(new_mms, axis=1)

    @pl.when(j == NBLK - 1)
    def _emit():
        cidx_ref[...] = jnp.concatenate(new_inds, axis=1)
        cmm_ref[...] = jnp.concatenate(new_mms, axis=1)


_short_call = pl.pallas_call(
    _short_kernel,
    grid=(NBLK,),
    in_specs=[
        pl.BlockSpec((MPAD, 3), lambda j: (0, 0)),
        pl.BlockSpec((MPAD, 1), lambda j: (0, 0)),
        pl.BlockSpec((3, BLK), lambda j: (0, j)),
        pl.BlockSpec((1, BLK), lambda j: (0, j)),
    ],
    out_specs=[
        pl.BlockSpec((MPAD, C), lambda j: (0, 0)),
        pl.BlockSpec((MPAD, C), lambda j: (0, 0)),
    ],
    out_shape=[
        jax.ShapeDtypeStruct((MPAD, C), jnp.int32),
        jax.ShapeDtypeStruct((MPAD, C), jnp.float32),
    ],
    scratch_shapes=[
        pltpu.VMEM((MPAD, C), jnp.float32),
        pltpu.VMEM((MPAD, C), jnp.int32),
        pltpu.VMEM((MPAD, C), jnp.float32),
    ],
    compiler_params=pltpu.CompilerParams(
        dimension_semantics=("arbitrary",),
    ),
)


_NW = 32
_BG = MPAD * C
_BPW = _BG // _NW
_D = 128


@functools.lru_cache(maxsize=1)
def _make_gather_call():
    mesh = plsc.VectorSubcoreMesh(core_axis_name="c", subcore_axis_name="s")

    @functools.partial(
        pl.kernel,
        out_type=jax.ShapeDtypeStruct((_BG, _D), jnp.float32),
        mesh=mesh,
        scratch_types=[
            pltpu.VMEM((_BPW,), jnp.int32),
            pltpu.VMEM((_BPW, _D), jnp.float32),
            pltpu.SemaphoreType.DMA,
        ],
    )
    def _gather_call(table_hbm, idx_hbm, out_hbm, idx_v, rows_v, sem):
        wid = lax.axis_index("s") * 2 + lax.axis_index("c")
        base = wid * _BPW
        pltpu.sync_copy(idx_hbm.at[pl.ds(base, _BPW)], idx_v)
        pltpu.async_copy(table_hbm.at[idx_v], rows_v, sem).wait()
        pltpu.sync_copy(rows_v, out_hbm.at[pl.ds(base, _BPW)])

    return _gather_call


def _asm_kernel(qn_ref, ci_ref, cm_ref, cp_ref, cand_ref, out_ref):
    qn = qn_ref[...]
    for b in range(QB):
        ci = ci_ref[b].reshape(1, C)
        cm = cm_ref[b].reshape(1, C)
        cp = cp_ref[b].reshape(1, C)
        cand = cand_ref[b]
        d = (qn - 2.0 * cm) + cp
        ohs = []
        for r in range(K):
            m = jnp.min(d, axis=1, keepdims=True)
            t = jnp.where(d == m, ci, IBIG)
            sel = jnp.min(t, axis=1, keepdims=True)
            hit = t == sel
            ohs.append(hit.astype(jnp.float32))
            d = jnp.where(hit, MASKV, d)
        oh = jnp.concatenate(ohs, axis=0)
        rows = jnp.dot(oh, cand, precision=jax.lax.Precision.HIGHEST)
        for r in range(K):
            out_ref[:, b, r, :] = rows[r * MPAD:r * MPAD + M]


_asm_call = pl.pallas_call(
    _asm_kernel,
    grid=(M // QB,),
    in_specs=[
        pl.BlockSpec((MPAD, 1), lambda q: (0, 0)),
        pl.BlockSpec((QB, 1, C), lambda q: (q, 0, 0)),
        pl.BlockSpec((QB, 1, C), lambda q: (q, 0, 0)),
        pl.BlockSpec((QB, 1, C), lambda q: (q, 0, 0)),
        pl.BlockSpec((QB, C, _D), lambda q: (q, 0, 0)),
    ],
    out_specs=pl.BlockSpec((M, QB, K, _D), lambda q: (0, q, 0, 0)),
    out_shape=jax.ShapeDtypeStruct((M, M, K, _D), jnp.float32),
    compiler_params=pltpu.CompilerParams(
        dimension_semantics=("arbitrary",),
    ),
)


def kernel(feats):
    p = feats[:, :3]
    n_p = p[:M]
    qn = jnp.sum(n_p[:, None, :] ** 2, axis=-1, keepdims=True)
    pn = jnp.sum(p ** 2, axis=-1)

    q_pad = jnp.zeros((MPAD, 3), jnp.float32).at[:M].set(n_p)
    qn_pad = jnp.zeros((MPAD, 1), jnp.float32).at[:M].set(qn.reshape(M, 1))
    pt_pad = jnp.pad(p.T, ((0, 0), (0, NPAD - N)))
    pn_pad = jnp.pad(pn, (0, NPAD - N), constant_values=PADV).reshape(1, NPAD)

    cidx, cmm = _short_call(q_pad, qn_pad, pt_pad, pn_pad)

    rows = _make_gather_call()(feats, cidx.reshape(_BG))
    cand = rows.reshape(MPAD, C, _D)
    cand = jnp.concatenate(
        [cand[:, :, :3] - q_pad[:, None, :], cand[:, :, 3:]], axis=-1)

    cpn = jnp.take(pn, cidx)
    out = _asm_call(qn_pad, cidx[:, None, :], cmm[:, None, :],
                    cpn[:, None, :], cand)
    return out


# speedup vs baseline: 1281.2223x; 1.3051x over previous
import functools

import jax
import jax.numpy as jnp
from jax import lax
from jax.experimental import pallas as pl
from jax.experimental.pallas import tpu as pltpu
from jax.experimental.pallas import tpu_sc as plsc

N = 50000
NPAD = 50176
M = 250
MPAD = 256
K = 16
C = 24
QB = 5
BLK = 7168
NBLK = NPAD // BLK
PADV = 1e30
MASKV = 2e30
IBIG = 2**30


def _short_kernel(q_ref, qn_ref, pt_ref, pn_ref, cidx_ref, cmm_ref,
                  vals_ref, inds_ref, mms_ref):
    j = pl.program_id(0)

    @pl.when(j == 0)
    def _init():
        vals_ref[...] = jnp.full((MPAD, C), MASKV, jnp.float32)
        inds_ref[...] = jnp.zeros((MPAD, C), jnp.int32)
        mms_ref[...] = jnp.zeros((MPAD, C), jnp.float32)

    q = q_ref[...]
    qn = qn_ref[...]
    pt = pt_ref[...]
    pn = pn_ref[...]
    mm = jnp.dot(q, pt)
    d = (qn - 2.0 * mm) + pn
    cols = lax.broadcasted_iota(jnp.int32, (MPAD, BLK), 1) + j * BLK

    svals = vals_ref[...]
    sinds = inds_ref[...]
    smms = mms_ref[...]

    new_vals, new_inds, new_mms = [], [], []
    for _ in range(C):
        m_d = jnp.min(d, axis=1, keepdims=True)
        m_s = jnp.min(svals, axis=1, keepdims=True)
        m = jnp.minimum(m_d, m_s)
        t_d = jnp.where(d == m, cols, IBIG)
        t_s = jnp.where(svals == m, sinds, IBIG)
        c_d = jnp.min(t_d, axis=1, keepdims=True)
        c_s = jnp.min(t_s, axis=1, keepdims=True)
        sel = jnp.minimum(c_d, c_s)
        hit_d = t_d == sel
        hit_s = t_s == sel
        mm_d = jnp.min(jnp.where(hit_d, mm, MASKV), axis=1, keepdims=True)
        mm_s = jnp.min(jnp.where(hit_s, smms, MASKV), axis=1, keepdims=True)
        new_vals.append(m)
        new_inds.append(sel)
        new_mms.append(jnp.minimum(mm_d, mm_s))
        d = jnp.where(hit_d, MASKV, d)
        svals = jnp.where(hit_s, MASKV, svals)

    vals_ref[...] = jnp.concatenate(new_vals, axis=1)
    inds_ref[...] = jnp.concatenate(new_inds, axis=1)
    mms_ref[...] = jnp.concatenate(new_mms, axis=1)

    @pl.when(j == NBLK - 1)
    def _emit():
        cidx_ref[...] = jnp.concatenate(new_inds, axis=1)
        cmm_ref[...] = jnp.concatenate(new_mms, axis=1)


_short_call = pl.pallas_call(
    _short_kernel,
    grid=(NBLK,),
    in_specs=[
        pl.BlockSpec((MPAD, 3), lambda j: (0, 0)),
        pl.BlockSpec((MPAD, 1), lambda j: (0, 0)),
        pl.BlockSpec((3, BLK), lambda j: (0, j)),
        pl.BlockSpec((1, BLK), lambda j: (0, j)),
    ],
    out_specs=[
        pl.BlockSpec((MPAD, C), lambda j: (0, 0)),
        pl.BlockSpec((MPAD, C), lambda j: (0, 0)),
    ],
    out_shape=[
        jax.ShapeDtypeStruct((MPAD, C), jnp.int32),
        jax.ShapeDtypeStruct((MPAD, C), jnp.float32),
    ],
    scratch_shapes=[
        pltpu.VMEM((MPAD, C), jnp.float32),
        pltpu.VMEM((MPAD, C), jnp.int32),
        pltpu.VMEM((MPAD, C), jnp.float32),
    ],
    compiler_params=pltpu.CompilerParams(
        dimension_semantics=("arbitrary",),
    ),
)


_NW = 32
_BG = MPAD * C
_BPW = _BG // _NW
_D = 128


@functools.lru_cache(maxsize=1)
def _make_gather_call():
    mesh = plsc.VectorSubcoreMesh(core_axis_name="c", subcore_axis_name="s")

    @functools.partial(
        pl.kernel,
        out_type=jax.ShapeDtypeStruct((_BG, _D), jnp.float32),
        mesh=mesh,
        scratch_types=[
            pltpu.VMEM((_BPW,), jnp.int32),
            pltpu.VMEM((_BPW, _D), jnp.float32),
            pltpu.SemaphoreType.DMA,
        ],
    )
    def _gather_call(table_hbm, idx_hbm, out_hbm, idx_v, rows_v, sem):
        wid = lax.axis_index("s") * 2 + lax.axis_index("c")
        base = wid * _BPW
        pltpu.sync_copy(idx_hbm.at[pl.ds(base, _BPW)], idx_v)
        pltpu.async_copy(table_hbm.at[idx_v], rows_v, sem).wait()
        pltpu.sync_copy(rows_v, out_hbm.at[pl.ds(base, _BPW)])

    return _gather_call


def _asm_kernel(qn_ref, ci_ref, cm_ref, cp_ref, cand_ref, out_ref):
    qn = qn_ref[...]
    for b in range(QB):
        ci = ci_ref[b]
        cm = cm_ref[b]
        cp = cp_ref[b]
        cand = cand_ref[b]
        d = (qn - 2.0 * cm) + cp
        for r in range(K):
            m = jnp.min(d, axis=0, keepdims=True)
            t = jnp.where(d == m, ci, IBIG)
            sel = jnp.min(t, axis=0, keepdims=True)
            hit = t == sel
            oh = hit.astype(jnp.float32)
            rows = jax.lax.dot_general(
                oh, cand, (((0,), (0,)), ((), ())),
                precision=jax.lax.Precision.HIGHEST)
            out_ref[:, b, r, :] = rows[:M]
            d = jnp.where(hit, MASKV, d)


_asm_call = pl.pallas_call(
    _asm_kernel,
    grid=(M // QB,),
    in_specs=[
        pl.BlockSpec((1, MPAD), lambda q: (0, 0)),
        pl.BlockSpec((QB, C, 1), lambda q: (q, 0, 0)),
        pl.BlockSpec((QB, C, 1), lambda q: (q, 0, 0)),
        pl.BlockSpec((QB, C, 1), lambda q: (q, 0, 0)),
        pl.BlockSpec((QB, C, _D), lambda q: (q, 0, 0)),
    ],
    out_specs=pl.BlockSpec((M, QB, K, _D), lambda q: (0, q, 0, 0)),
    out_shape=jax.ShapeDtypeStruct((M, M, K, _D), jnp.float32),
    compiler_params=pltpu.CompilerParams(
        dimension_semantics=("arbitrary",),
    ),
)


def kernel(feats):
    p = feats[:, :3]
    n_p = p[:M]
    qn = jnp.sum(n_p[:, None, :] ** 2, axis=-1, keepdims=True)
    pn = jnp.sum(p ** 2, axis=-1)

    q_pad = jnp.zeros((MPAD, 3), jnp.float32).at[:M].set(n_p)
    qn_pad = jnp.zeros((MPAD, 1), jnp.float32).at[:M].set(qn.reshape(M, 1))
    pt_pad = jnp.pad(p.T, ((0, 0), (0, NPAD - N)))
    pn_pad = jnp.pad(pn, (0, NPAD - N), constant_values=PADV).reshape(1, NPAD)

    cidx, cmm = _short_call(q_pad, qn_pad, pt_pad, pn_pad)

    rows = _make_gather_call()(feats, cidx.reshape(_BG))
    cand = rows.reshape(MPAD, C, _D)
    cand = jnp.concatenate(
        [cand[:, :, :3] - q_pad[:, None, :], cand[:, :, 3:]], axis=-1)

    cpn = jnp.take(pn, cidx)
    out = _asm_call(qn_pad.reshape(1, MPAD), cidx[:, :, None],
                    cmm[:, :, None], cpn[:, :, None], cand)
    return out
